# Initial kernel scaffold; baseline (speedup 1.0000x reference)
import jax
import jax.numpy as jnp
from jax.experimental import pallas as pl


def kernel(x):
    raise NotImplementedError("write your pallas kernel here")


# speedup vs baseline: 14.3169x; 14.3169x over previous
import jax
import jax.numpy as jnp
from jax.experimental import pallas as pl
from jax.experimental.pallas import tpu as pltpu

_K = 32
_MASK31 = 0x7FFFFFFF
_INT_MIN = -2147483648


def _sparsify_body(x_ref, o_ref):
    x = x_ref[...]
    i = jax.lax.bitcast_convert_type(x, jnp.int32)
    key = jnp.where(i >= 0, i, i ^ jnp.int32(_MASK31))

    rows = x.shape[0]
    tu = jnp.zeros((rows, 1), jnp.int32)

    def body(b, tu):
        bit = jax.lax.shift_left(jnp.int32(1), 31 - b)
        cand = tu | bit
        thr = cand ^ jnp.int32(_INT_MIN)
        cnt = jnp.sum((key >= thr).astype(jnp.int32), axis=1, keepdims=True)
        return jnp.where(cnt >= _K, cand, tu)

    tu = jax.lax.fori_loop(0, 32, body, tu, unroll=True)

    kth_key = tu ^ jnp.int32(_INT_MIN)
    kth_bits = jnp.where(kth_key >= 0, kth_key, kth_key ^ jnp.int32(_MASK31))
    kth = jax.lax.bitcast_convert_type(kth_bits, jnp.float32)
    o_ref[...] = jnp.where(x >= kth, x, jnp.float32(0.0))


def kernel(x):
    return pl.pallas_call(
        _sparsify_body,
        out_shape=jax.ShapeDtypeStruct(x.shape, x.dtype),
    )(x)


# speedup vs baseline: 21.7908x; 1.5220x over previous
import jax
import jax.numpy as jnp
from jax.experimental import pallas as pl
from jax.experimental.pallas import tpu as pltpu

_K = 32
_MASK31 = 0x7FFFFFFF
_INT_MIN = -2147483648
_BIG = 3.0e38


def _keys_of(x):
    i = jax.lax.bitcast_convert_type(x, jnp.int32)
    return jnp.where(i >= 0, i, i ^ jnp.int32(_MASK31))


def _kth_largest_key(key, k):
    rows = key.shape[0]
    tu = jnp.zeros((rows, 1), jnp.int32)

    def body(b, tu):
        bit = jax.lax.shift_left(jnp.int32(1), 31 - b)
        cand = tu | bit
        thr = cand ^ jnp.int32(_INT_MIN)
        cnt = jnp.sum((key >= thr).astype(jnp.int32), axis=1, keepdims=True)
        return jnp.where(cnt >= k, cand, tu)

    tu = jax.lax.fori_loop(0, 32, body, tu, unroll=True)
    return tu ^ jnp.int32(_INT_MIN)


def _sparsify_body(x_ref, o_ref):
    x = x_ref[...]
    key = _keys_of(x)

    m = key[:, 0:256]
    for e in range(1, 32):
        m = jnp.maximum(m, key[:, e * 256:(e + 1) * 256])

    lkey = _kth_largest_key(m, _K)

    flags = (m > lkey).astype(jnp.float32)
    ri = jax.lax.broadcasted_iota(jnp.int32, (256, 256), 0)
    ci = jax.lax.broadcasted_iota(jnp.int32, (256, 256), 1)
    tri = (ri <= ci).astype(jnp.float32)
    ranks = jax.lax.dot_general(
        flags, tri, (((1,), (0,)), ((), ())),
        preferred_element_type=jnp.float32)

    ranks_i = ranks.astype(jnp.int32)
    jj = jax.lax.broadcasted_iota(jnp.int32, (1, _K, 1), 1) + 1
    sel = jnp.where(
        (ranks_i[:, None, :] == jj) & (flags[:, None, :] > 0.0),
        1.0, 0.0).astype(jnp.float32)

    x3 = x.reshape(128, 32, 256)
    buf = jax.lax.dot_general(
        sel, x3, (((2,), (2,)), ((0,), (0,))),
        preferred_element_type=jnp.float32)
    got = jnp.sum(sel, axis=2)
    buf = buf + (got[:, :, None] - 1.0) * jnp.float32(_BIG)

    bkey = _keys_of(buf.reshape(128, 32 * _K))
    tkey = _kth_largest_key(bkey, _K)

    kth_key = jnp.maximum(lkey, tkey)
    kth_bits = jnp.where(kth_key >= 0, kth_key, kth_key ^ jnp.int32(_MASK31))
    kth = jax.lax.bitcast_convert_type(kth_bits, jnp.float32)
    o_ref[...] = jnp.where(x >= kth, x, jnp.float32(0.0))


def kernel(x):
    return pl.pallas_call(
        _sparsify_body,
        out_shape=jax.ShapeDtypeStruct(x.shape, x.dtype),
    )(x)
